# baseline (speedup 1.0000x reference)
import jax
import jax.numpy as jnp
from jax import lax
from jax.experimental import pallas as pl
from jax.experimental.pallas import tpu as pltpu
from jax.experimental.pallas import tpu_sc as plsc

N_VERT = 100000
SIZES = (50000, 16, 20000, 2)
TOTAL = sum(SIZES)

NC, NS = 2, 16
NW = NC * NS
CHUNK = 128
K = -(-TOTAL // (NW * CHUNK))
PAD_TOTAL = NW * K * CHUNK

def _gather_body(table_hbm, idx_hbm, out_hbm, idx_v, rows_v, sem):
    wid = lax.axis_index("s") * NC + lax.axis_index("c")
    pltpu.sync_copy(idx_hbm.at[wid], idx_v)
    copies = [
        pltpu.async_copy(table_hbm.at[idx_v.at[j]], rows_v.at[j], sem)
        for j in range(K)
    ]
    for c in copies:
        c.wait()
    pltpu.sync_copy(rows_v, out_hbm.at[wid])


@jax.jit
def kernel(filtration, finite_idx_0, essential_idx_0, finite_idx_1, essential_idx_1):
    idx_all = jnp.concatenate(
        [finite_idx_0, essential_idx_0, finite_idx_1, essential_idx_1]
    )
    pad = jnp.arange(PAD_TOTAL - TOTAL, dtype=jnp.int32) % N_VERT
    idx_all = jnp.concatenate([idx_all, pad]).reshape(NW, K, CHUNK)

    gathered = pl.kernel(
        _gather_body,
        out_type=jax.ShapeDtypeStruct((NW, K, CHUNK), jnp.float32),
        mesh=plsc.VectorSubcoreMesh(
            core_axis_name="c", subcore_axis_name="s", num_cores=NC, num_subcores=NS
        ),
        scratch_types=[
            pltpu.VMEM((K, CHUNK), jnp.int32),
            pltpu.VMEM((K, CHUNK), jnp.float32),
            pltpu.SemaphoreType.DMA,
        ],
    )(filtration, idx_all)

    flat = gathered.reshape(-1)
    o0, o1, o2, o3 = SIZES
    finite_dgm_0 = flat[:o0].reshape(-1, 2)
    essential_dgm_0 = flat[o0 : o0 + o1].reshape(-1, 1)
    finite_dgm_1 = flat[o0 + o1 : o0 + o1 + o2].reshape(-1, 2)
    essential_dgm_1 = flat[o0 + o1 + o2 : TOTAL].reshape(-1, 1)
    return (finite_dgm_0, essential_dgm_0, finite_dgm_1, essential_dgm_1)


# speedup vs baseline: 1.2847x; 1.2847x over previous
import jax
import jax.numpy as jnp
from jax import lax
from jax.experimental import pallas as pl
from jax.experimental.pallas import tpu as pltpu
from jax.experimental.pallas import tpu_sc as plsc

N_VERT = 100000
N_F0, N_E0, N_F1, N_E1 = 50000, 16, 20000, 2

NC, NS = 2, 16
NW = NC * NS
CHUNK = 128


def _shard(n):
    s = -(-n // NW)
    return -(-s // 8) * 8


S0 = _shard(N_F0)
S1 = _shard(N_F1)


def _chunks(total):
    return [(st, min(CHUNK, total - st)) for st in range(0, total, CHUNK)]


def _gather_body(table, f0, e0, f1, e1, o_f0, o_e0, o_f1, o_e1,
                 idx0_v, rows0_v, idx1_v, rows1_v, idxe0_v, rowse0_v,
                 idxe1_v, rowse1_v, sem, sem_e):
    wid = lax.axis_index("s") * NC + lax.axis_index("c")

    off0 = jnp.minimum(wid * S0, N_F0 - S0)
    off1 = jnp.minimum(wid * S1, N_F1 - S1)
    pltpu.sync_copy(f0.at[pl.ds(off0, S0)], idx0_v)
    pltpu.sync_copy(f1.at[pl.ds(off1, S1)], idx1_v)

    copies = []
    for st, sz in _chunks(S0):
        copies.append(pltpu.async_copy(
            table.at[idx0_v.at[pl.ds(st, sz)]], rows0_v.at[pl.ds(st, sz)], sem))
    for st, sz in _chunks(S1):
        copies.append(pltpu.async_copy(
            table.at[idx1_v.at[pl.ds(st, sz)]], rows1_v.at[pl.ds(st, sz)], sem))

    @pl.when(wid == 0)
    def _essentials():
        pltpu.sync_copy(e0, idxe0_v)
        pltpu.sync_copy(e1, idxe1_v)
        pltpu.async_copy(table.at[idxe0_v], rowse0_v, sem_e).wait()
        pltpu.async_copy(table.at[idxe1_v], rowse1_v, sem_e).wait()
        pltpu.sync_copy(rowse0_v, o_e0)
        pltpu.sync_copy(rowse1_v, o_e1)

    for c in copies:
        c.wait()
    pltpu.sync_copy(rows0_v, o_f0.at[pl.ds(off0, S0)])
    pltpu.sync_copy(rows1_v, o_f1.at[pl.ds(off1, S1)])


@jax.jit
def kernel(filtration, finite_idx_0, essential_idx_0, finite_idx_1, essential_idx_1):
    f0, e0, f1, e1 = pl.kernel(
        _gather_body,
        out_type=(
            jax.ShapeDtypeStruct((N_F0,), jnp.float32),
            jax.ShapeDtypeStruct((N_E0,), jnp.float32),
            jax.ShapeDtypeStruct((N_F1,), jnp.float32),
            jax.ShapeDtypeStruct((N_E1,), jnp.float32),
        ),
        mesh=plsc.VectorSubcoreMesh(
            core_axis_name="c", subcore_axis_name="s", num_cores=NC, num_subcores=NS
        ),
        scratch_types=[
            pltpu.VMEM((S0,), jnp.int32),
            pltpu.VMEM((S0,), jnp.float32),
            pltpu.VMEM((S1,), jnp.int32),
            pltpu.VMEM((S1,), jnp.float32),
            pltpu.VMEM((N_E0,), jnp.int32),
            pltpu.VMEM((N_E0,), jnp.float32),
            pltpu.VMEM((N_E1,), jnp.int32),
            pltpu.VMEM((N_E1,), jnp.float32),
            pltpu.SemaphoreType.DMA,
            pltpu.SemaphoreType.DMA,
        ],
    )(filtration, finite_idx_0, essential_idx_0, finite_idx_1, essential_idx_1)

    return (
        f0.reshape(-1, 2),
        e0.reshape(-1, 1),
        f1.reshape(-1, 2),
        e1.reshape(-1, 1),
    )


# speedup vs baseline: 1.2848x; 1.0001x over previous
import functools

import jax
import jax.numpy as jnp
from jax import lax
from jax.experimental import pallas as pl
from jax.experimental.layout import Format, Layout
from jax.experimental.pallas import tpu as pltpu
from jax.experimental.pallas import tpu_sc as plsc

N_VERT = 100000
N_F0, N_E0, N_F1, N_E1 = 50000, 16, 20000, 2

NC, NS = 2, 16
NW = NC * NS
CHUNK = 128


def _shard(n):
    s = -(-n // NW)
    return -(-s // 8) * 8


S0 = _shard(N_F0)
S1 = _shard(N_F1)


def _chunks(total):
    return [(st, min(CHUNK, total - st)) for st in range(0, total, CHUNK)]


def _gather_body(table, f0, e0, f1, e1, o_f0, o_e0, o_f1, o_e1,
                 idx0_v, rows0_v, idx1_v, rows1_v, idxe0_v, rowse0_v,
                 idxe1_v, rowse1_v, sem, sem_e):
    wid = lax.axis_index("s") * NC + lax.axis_index("c")

    off0 = jnp.minimum(wid * S0, N_F0 - S0)
    off1 = jnp.minimum(wid * S1, N_F1 - S1)
    pltpu.sync_copy(f0.at[pl.ds(off0, S0)], idx0_v)
    pltpu.sync_copy(f1.at[pl.ds(off1, S1)], idx1_v)

    copies = []
    for st, sz in _chunks(S0):
        copies.append(pltpu.async_copy(
            table.at[idx0_v.at[pl.ds(st, sz)]], rows0_v.at[pl.ds(st, sz)], sem))
    for st, sz in _chunks(S1):
        copies.append(pltpu.async_copy(
            table.at[idx1_v.at[pl.ds(st, sz)]], rows1_v.at[pl.ds(st, sz)], sem))

    @pl.when(wid == 0)
    def _essentials():
        pltpu.sync_copy(e0, idxe0_v)
        pltpu.sync_copy(e1, idxe1_v)
        pltpu.async_copy(table.at[idxe0_v], rowse0_v, sem_e).wait()
        pltpu.async_copy(table.at[idxe1_v], rowse1_v, sem_e).wait()
        pltpu.sync_copy(rowse0_v, o_e0)
        pltpu.sync_copy(rowse1_v, o_e1)

    for c in copies:
        c.wait()
    pltpu.sync_copy(rows0_v, o_f0.at[pl.ds(off0, S0)])
    pltpu.sync_copy(rows1_v, o_f1.at[pl.ds(off1, S1)])


_LINEAR2D = Layout(major_to_minor=(0, 1), tiling=((8, 1),))


@functools.cache
def _jitted():
    fmt = Format(_LINEAR2D, jax.sharding.SingleDeviceSharding(jax.devices()[0]))
    return jax.jit(_kernel_impl, out_shardings=(fmt,) * 4)


def kernel(filtration, finite_idx_0, essential_idx_0, finite_idx_1, essential_idx_1):
    return _jitted()(
        filtration, finite_idx_0, essential_idx_0, finite_idx_1, essential_idx_1
    )


def _kernel_impl(filtration, finite_idx_0, essential_idx_0, finite_idx_1, essential_idx_1):
    f0, e0, f1, e1 = pl.kernel(
        _gather_body,
        out_type=(
            jax.ShapeDtypeStruct((N_F0,), jnp.float32),
            jax.ShapeDtypeStruct((N_E0,), jnp.float32),
            jax.ShapeDtypeStruct((N_F1,), jnp.float32),
            jax.ShapeDtypeStruct((N_E1,), jnp.float32),
        ),
        mesh=plsc.VectorSubcoreMesh(
            core_axis_name="c", subcore_axis_name="s", num_cores=NC, num_subcores=NS
        ),
        scratch_types=[
            pltpu.VMEM((S0,), jnp.int32),
            pltpu.VMEM((S0,), jnp.float32),
            pltpu.VMEM((S1,), jnp.int32),
            pltpu.VMEM((S1,), jnp.float32),
            pltpu.VMEM((N_E0,), jnp.int32),
            pltpu.VMEM((N_E0,), jnp.float32),
            pltpu.VMEM((N_E1,), jnp.int32),
            pltpu.VMEM((N_E1,), jnp.float32),
            pltpu.SemaphoreType.DMA,
            pltpu.SemaphoreType.DMA,
        ],
    )(filtration, finite_idx_0, essential_idx_0, finite_idx_1, essential_idx_1)

    return (
        f0.reshape(-1, 2),
        e0.reshape(-1, 1),
        f1.reshape(-1, 2),
        e1.reshape(-1, 1),
    )


# speedup vs baseline: 1.7398x; 1.3541x over previous
import jax
import jax.numpy as jnp
from jax import lax
from jax.experimental import pallas as pl
from jax.experimental.pallas import tpu as pltpu
from jax.experimental.pallas import tpu_sc as plsc

N_VERT = 100000
N_F0, N_E0, N_F1, N_E1 = 50000, 16, 20000, 2

NC, NS = 2, 16
NW = NC * NS
CHUNK = 128


def _shard(n):
    s = -(-n // NW)
    return -(-s // 8) * 8


S0 = _shard(N_F0)
S1 = _shard(N_F1)


def _chunks(total):
    return [(st, min(CHUNK, total - st)) for st in range(0, total, CHUNK)]


def _gather_body(table, f0, e0, f1, e1, o_f0, o_e0, o_f1, o_e1,
                 idx0_v, rows0_v, idx1_v, rows1_v, idxe0_v, rowse0_v,
                 idxe1_v, rowse1_v, sem, sem_e):
    wid = lax.axis_index("s") * NC + lax.axis_index("c")

    off0 = jnp.minimum(wid * S0, N_F0 - S0)
    off1 = jnp.minimum(wid * S1, N_F1 - S1)
    pltpu.sync_copy(f0.at[pl.ds(off0, S0)], idx0_v)
    pltpu.sync_copy(f1.at[pl.ds(off1, S1)], idx1_v)

    copies = []
    for st, sz in _chunks(S0):
        copies.append(pltpu.async_copy(
            table.at[idx0_v.at[pl.ds(st, sz)]], rows0_v.at[pl.ds(st, sz)], sem))
    for st, sz in _chunks(S1):
        copies.append(pltpu.async_copy(
            table.at[idx1_v.at[pl.ds(st, sz)]], rows1_v.at[pl.ds(st, sz)], sem))

    @pl.when(wid == 0)
    def _essentials():
        pltpu.sync_copy(e0, idxe0_v)
        pltpu.sync_copy(e1, idxe1_v)
        pltpu.async_copy(table.at[idxe0_v], rowse0_v, sem_e).wait()
        pltpu.async_copy(table.at[idxe1_v], rowse1_v, sem_e).wait()
        pltpu.sync_copy(rowse0_v, o_e0)
        pltpu.sync_copy(rowse1_v, o_e1)

    for c in copies:
        c.wait()
    pltpu.sync_copy(rows0_v, o_f0.at[pl.ds(off0, S0)])
    pltpu.sync_copy(rows1_v, o_f1.at[pl.ds(off1, S1)])


@jax.jit
def kernel(filtration, finite_idx_0, essential_idx_0, finite_idx_1, essential_idx_1):
    f0, e0, f1, e1 = pl.kernel(
        _gather_body,
        out_type=(
            jax.ShapeDtypeStruct((N_F0,), jnp.float32),
            jax.ShapeDtypeStruct((N_E0,), jnp.float32),
            jax.ShapeDtypeStruct((N_F1,), jnp.float32),
            jax.ShapeDtypeStruct((N_E1,), jnp.float32),
        ),
        mesh=plsc.VectorSubcoreMesh(
            core_axis_name="c", subcore_axis_name="s", num_cores=NC, num_subcores=NS
        ),
        scratch_types=[
            pltpu.VMEM((S0,), jnp.int32),
            pltpu.VMEM((S0,), jnp.float32),
            pltpu.VMEM((S1,), jnp.int32),
            pltpu.VMEM((S1,), jnp.float32),
            pltpu.VMEM((N_E0,), jnp.int32),
            pltpu.VMEM((N_E0,), jnp.float32),
            pltpu.VMEM((N_E1,), jnp.int32),
            pltpu.VMEM((N_E1,), jnp.float32),
            pltpu.SemaphoreType.DMA,
            pltpu.SemaphoreType.DMA,
        ],
    )(filtration, finite_idx_0, essential_idx_0, finite_idx_1, essential_idx_1)

    return (
        jnp.stack([f0[0::2], f0[1::2]], axis=1),
        e0.reshape(-1, 1),
        jnp.stack([f1[0::2], f1[1::2]], axis=1),
        e1.reshape(-1, 1),
    )


# speedup vs baseline: 2.2023x; 1.2658x over previous
import jax
import jax.numpy as jnp
from jax import lax
from jax.experimental import pallas as pl
from jax.experimental.pallas import tpu as pltpu
from jax.experimental.pallas import tpu_sc as plsc

N_VERT = 100000
N_F0, N_E0, N_F1, N_E1 = 50000, 16, 20000, 2
P0, P1 = N_F0 // 2, N_F1 // 2

NC, NS = 2, 16
NW = NC * NS
L = 16
CHUNK = 128


def _shard(n):
    s = -(-n // NW)
    return -(-s // L) * L


SP0 = _shard(P0)
SP1 = _shard(P1)


def _chunks(total):
    return [(st, min(CHUNK, total - st)) for st in range(0, total, CHUNK)]


def _fill_positions(pos_v, base, size, parity):
    lane2 = lax.iota(jnp.int32, L) * 2
    start = base * 2 + parity
    for v in range(size // L):
        pos_v[pl.ds(v * L, L)] = lane2 + (start + v * 2 * L)


def _gather_body(table, f0, e0, f1, e1,
                 o_b0, o_d0, o_e0, o_b1, o_d1, o_e1,
                 pb0_v, pd0_v, ib0_v, id0_v, b0_v, d0_v,
                 pb1_v, pd1_v, ib1_v, id1_v, b1_v, d1_v,
                 idxe0_v, rowse0_v, idxe1_v, rowse1_v, sem, sem_e):
    wid = lax.axis_index("s") * NC + lax.axis_index("c")

    off0 = jnp.minimum(wid * SP0, P0 - SP0)
    off1 = jnp.minimum(wid * SP1, P1 - SP1)

    _fill_positions(pb0_v, off0, SP0, 0)
    _fill_positions(pd0_v, off0, SP0, 1)
    _fill_positions(pb1_v, off1, SP1, 0)
    _fill_positions(pd1_v, off1, SP1, 1)

    copies = []
    for src, pos_v, idx_v, sp in ((f0, pb0_v, ib0_v, SP0), (f0, pd0_v, id0_v, SP0),
                                  (f1, pb1_v, ib1_v, SP1), (f1, pd1_v, id1_v, SP1)):
        for st, sz in _chunks(sp):
            copies.append(pltpu.async_copy(
                src.at[pos_v.at[pl.ds(st, sz)]], idx_v.at[pl.ds(st, sz)], sem))
    for c in copies:
        c.wait()

    copies = []
    for idx_v, rows_v, sp in ((ib0_v, b0_v, SP0), (id0_v, d0_v, SP0),
                              (ib1_v, b1_v, SP1), (id1_v, d1_v, SP1)):
        for st, sz in _chunks(sp):
            copies.append(pltpu.async_copy(
                table.at[idx_v.at[pl.ds(st, sz)]], rows_v.at[pl.ds(st, sz)],
                sem))

    @pl.when(wid == 0)
    def _essentials():
        pltpu.sync_copy(e0, idxe0_v)
        pltpu.sync_copy(e1, idxe1_v)
        pltpu.async_copy(table.at[idxe0_v], rowse0_v, sem_e).wait()
        pltpu.async_copy(table.at[idxe1_v], rowse1_v, sem_e).wait()
        pltpu.sync_copy(rowse0_v, o_e0)
        pltpu.sync_copy(rowse1_v, o_e1)

    for c in copies:
        c.wait()

    pltpu.sync_copy(b0_v, o_b0.at[pl.ds(off0, SP0)])
    pltpu.sync_copy(d0_v, o_d0.at[pl.ds(off0, SP0)])
    pltpu.sync_copy(b1_v, o_b1.at[pl.ds(off1, SP1)])
    pltpu.sync_copy(d1_v, o_d1.at[pl.ds(off1, SP1)])


@jax.jit
def kernel(filtration, finite_idx_0, essential_idx_0, finite_idx_1, essential_idx_1):
    b0, d0, e0, b1, d1, e1 = pl.kernel(
        _gather_body,
        out_type=(
            jax.ShapeDtypeStruct((P0,), jnp.float32),
            jax.ShapeDtypeStruct((P0,), jnp.float32),
            jax.ShapeDtypeStruct((N_E0,), jnp.float32),
            jax.ShapeDtypeStruct((P1,), jnp.float32),
            jax.ShapeDtypeStruct((P1,), jnp.float32),
            jax.ShapeDtypeStruct((N_E1,), jnp.float32),
        ),
        mesh=plsc.VectorSubcoreMesh(
            core_axis_name="c", subcore_axis_name="s", num_cores=NC, num_subcores=NS
        ),
        scratch_types=[
            pltpu.VMEM((SP0,), jnp.int32),
            pltpu.VMEM((SP0,), jnp.int32),
            pltpu.VMEM((SP0,), jnp.int32),
            pltpu.VMEM((SP0,), jnp.int32),
            pltpu.VMEM((SP0,), jnp.float32),
            pltpu.VMEM((SP0,), jnp.float32),
            pltpu.VMEM((SP1,), jnp.int32),
            pltpu.VMEM((SP1,), jnp.int32),
            pltpu.VMEM((SP1,), jnp.int32),
            pltpu.VMEM((SP1,), jnp.int32),
            pltpu.VMEM((SP1,), jnp.float32),
            pltpu.VMEM((SP1,), jnp.float32),
            pltpu.VMEM((N_E0,), jnp.int32),
            pltpu.VMEM((N_E0,), jnp.float32),
            pltpu.VMEM((N_E1,), jnp.int32),
            pltpu.VMEM((N_E1,), jnp.float32),
            pltpu.SemaphoreType.DMA,
            pltpu.SemaphoreType.DMA,
        ],
    )(filtration, finite_idx_0, essential_idx_0, finite_idx_1, essential_idx_1)

    return (
        jnp.stack([b0, d0], axis=1),
        e0.reshape(-1, 1),
        jnp.stack([b1, d1], axis=1),
        e1.reshape(-1, 1),
    )
